# baseline (device time: 64983 ns/iter reference)
import jax
import jax.numpy as jnp
from jax import lax
from jax.experimental import pallas as pl
from jax.experimental.pallas import tpu as pltpu

N_DEV = 4
B, SQ, D_MODEL = 2, 256, 512
H_TOT, H_LOC, DH = 16, 4, 64
SKV_SH = 256
WIN = 128


def kernel(x, Wq, K_ext, V_ext, Wo):
    def body(x_ref, wq_ref, k_ref, v_ref, wo_ref, out_ref,
             kvs, kvr, cbuf, kv_send_sems, kv_recv_sems,
             ring_send_sems, ring_recv_sems):
        me = lax.axis_index("i")

        @pl.when(me < 2)
        def _():
            kvs[0] = jnp.transpose(
                k_ref[...].astype(jnp.bfloat16), (0, 2, 1, 3))
            kvs[1] = jnp.transpose(
                v_ref[...].astype(jnp.bfloat16), (0, 2, 1, 3))

        for src in range(2):
            @pl.when(me == src)
            def _(src=src):
                kvr[src] = kvs[:, :, 4 * src:4 * src + 4]
                for n, j in enumerate(d for d in range(N_DEV) if d != src):
                    rdma = pltpu.make_async_remote_copy(
                        src_ref=kvs.at[:, :, pl.ds(4 * j, H_LOC)],
                        dst_ref=kvr.at[src],
                        send_sem=kv_send_sems.at[n],
                        recv_sem=kv_recv_sems.at[src],
                        device_id=(j,),
                        device_id_type=pl.DeviceIdType.MESH,
                    )
                    rdma.start()

        xb = x_ref[...].astype(jnp.bfloat16)
        wqb = wq_ref[...].astype(jnp.bfloat16)

        for src in range(2):
            @pl.when(me != src)
            def _(src=src):
                recv = pltpu.make_async_remote_copy(
                    src_ref=kvs.at[:, :, 0:H_LOC],
                    dst_ref=kvr.at[src],
                    send_sem=kv_send_sems.at[0],
                    recv_sem=kv_recv_sems.at[src],
                    device_id=(src,),
                    device_id_type=pl.DeviceIdType.MESH,
                )
                recv.wait_recv()

        k_all = jnp.concatenate([kvr[0, 0], kvr[1, 0]], axis=2)
        v_all = jnp.concatenate([kvr[0, 1], kvr[1, 1]], axis=2)

        qi = lax.broadcasted_iota(jnp.int32, (SQ, 2 * SKV_SH), 0)
        ki = lax.broadcasted_iota(jnp.int32, (SQ, 2 * SKV_SH), 1)
        mask = jnp.abs(qi - ki) <= WIN

        for b in range(B):
            qb = jnp.dot(xb[b], wqb,
                         preferred_element_type=jnp.float32
                         ).astype(jnp.bfloat16)
            pb = jnp.zeros((SQ, D_MODEL), jnp.float32)
            for h in range(H_LOC):
                qh = qb[:, h * DH:(h + 1) * DH]
                s = lax.dot_general(
                    qh, k_all[b, h], (((1,), (1,)), ((), ())),
                    preferred_element_type=jnp.float32) * 0.125
                s = jnp.where(mask, s, jnp.float32(-1e9))
                m = jnp.max(s, axis=1, keepdims=True)
                w = jnp.exp(s - m)
                w = w / jnp.sum(w, axis=1, keepdims=True)
                ctx = jnp.dot(w.astype(jnp.bfloat16), v_all[b, h],
                              preferred_element_type=jnp.float32)
                wo_h = wo_ref[h * DH:(h + 1) * DH, :].astype(jnp.bfloat16)
                pb = pb + jnp.dot(ctx.astype(jnp.bfloat16), wo_h,
                                  preferred_element_type=jnp.float32)
            out_ref[b] = pb
            cbuf[0, b] = pb.astype(jnp.bfloat16)

        @pl.when(me < 2)
        def _():
            for n in range(N_DEV - 1):
                done = pltpu.make_async_remote_copy(
                    src_ref=kvs.at[:, :, 0:H_LOC],
                    dst_ref=kvr.at[0],
                    send_sem=kv_send_sems.at[n],
                    recv_sem=kv_recv_sems.at[0],
                    device_id=(0,),
                    device_id_type=pl.DeviceIdType.MESH,
                )
                done.wait_send()

        right = (me + 1) % N_DEV
        for hop in range(N_DEV - 1):
            rdma = pltpu.make_async_remote_copy(
                src_ref=cbuf.at[hop],
                dst_ref=cbuf.at[hop + 1],
                send_sem=ring_send_sems.at[hop],
                recv_sem=ring_recv_sems.at[hop],
                device_id=(right,),
                device_id_type=pl.DeviceIdType.MESH,
            )
            rdma.start()
            rdma.wait()
            out_ref[...] = out_ref[...] + cbuf[hop + 1].astype(jnp.float32)

    return pl.pallas_call(
        body,
        out_shape=jax.ShapeDtypeStruct((B, SQ, D_MODEL), jnp.float32),
        in_specs=[pl.BlockSpec(memory_space=pltpu.VMEM)] * 5,
        out_specs=pl.BlockSpec(memory_space=pltpu.VMEM),
        scratch_shapes=[
            pltpu.VMEM((2, B, H_TOT, SKV_SH, DH), jnp.bfloat16),
            pltpu.VMEM((2, 2, B, H_LOC, SKV_SH, DH), jnp.bfloat16),
            pltpu.VMEM((N_DEV, B, SQ, D_MODEL), jnp.bfloat16),
            pltpu.SemaphoreType.DMA((N_DEV - 1,)),
            pltpu.SemaphoreType.DMA((2,)),
            pltpu.SemaphoreType.DMA((N_DEV - 1,)),
            pltpu.SemaphoreType.DMA((N_DEV - 1,)),
        ],
    )(x, Wq, K_ext, V_ext, Wo)


# device time: 57331 ns/iter; 1.1335x vs baseline; 1.1335x over previous
import jax
import jax.numpy as jnp
from jax import lax
from jax.experimental import pallas as pl
from jax.experimental.pallas import tpu as pltpu

N_DEV = 4
B, SQ, D_MODEL = 2, 256, 512
H_TOT, H_LOC, DH = 16, 4, 64
SKV_SH = 256
WIN = 128
SKV1 = 128
KV_COLS = SKV_SH + SKV1


def kernel(x, Wq, K_ext, V_ext, Wo):
    def body(x_ref, wq_ref, k_ref, v_ref, wo_ref, out_ref,
             kvs, kvr0, kvr1, cbuf, kv_send_sems, kv_recv_sems,
             bf_send_sems, bf_recv_sems):
        me = lax.axis_index("i")

        @pl.when(me < 2)
        def _():
            kvs[0] = jnp.transpose(
                k_ref[...].astype(jnp.bfloat16), (0, 2, 1, 3))
            kvs[1] = jnp.transpose(
                v_ref[...].astype(jnp.bfloat16), (0, 2, 1, 3))

        @pl.when(me == 0)
        def _():
            kvr0[...] = kvs[:, :, 0:H_LOC]
            for n, j in enumerate((1, 2, 3)):
                rdma = pltpu.make_async_remote_copy(
                    src_ref=kvs.at[:, :, pl.ds(4 * j, H_LOC)],
                    dst_ref=kvr0,
                    send_sem=kv_send_sems.at[n],
                    recv_sem=kv_recv_sems.at[0],
                    device_id=(j,),
                    device_id_type=pl.DeviceIdType.MESH,
                )
                rdma.start()

        @pl.when(me == 1)
        def _():
            kvr1[...] = kvs[:, :, H_LOC:2 * H_LOC, 0:SKV1]
            for n, j in enumerate((0, 2, 3)):
                rdma = pltpu.make_async_remote_copy(
                    src_ref=kvs.at[:, :, pl.ds(4 * j, H_LOC), 0:SKV1],
                    dst_ref=kvr1,
                    send_sem=kv_send_sems.at[n],
                    recv_sem=kv_recv_sems.at[1],
                    device_id=(j,),
                    device_id_type=pl.DeviceIdType.MESH,
                )
                rdma.start()

        xb = x_ref[...].astype(jnp.bfloat16)
        wqb = wq_ref[...].astype(jnp.bfloat16)

        @pl.when(me != 0)
        def _():
            recv = pltpu.make_async_remote_copy(
                src_ref=kvs.at[:, :, 0:H_LOC],
                dst_ref=kvr0,
                send_sem=kv_send_sems.at[0],
                recv_sem=kv_recv_sems.at[0],
                device_id=(0,),
                device_id_type=pl.DeviceIdType.MESH,
            )
            recv.wait_recv()

        @pl.when(me != 1)
        def _():
            recv = pltpu.make_async_remote_copy(
                src_ref=kvs.at[:, :, 0:H_LOC, 0:SKV1],
                dst_ref=kvr1,
                send_sem=kv_send_sems.at[0],
                recv_sem=kv_recv_sems.at[1],
                device_id=(1,),
                device_id_type=pl.DeviceIdType.MESH,
            )
            recv.wait_recv()

        k_all = jnp.concatenate([kvr0[0], kvr1[0]], axis=2)
        v_all = jnp.concatenate([kvr0[1], kvr1[1]], axis=2)

        qi = lax.broadcasted_iota(jnp.int32, (SQ, KV_COLS), 0)
        ki = lax.broadcasted_iota(jnp.int32, (SQ, KV_COLS), 1)
        mask = jnp.abs(qi - ki) <= WIN

        for b in range(B):
            qb = jnp.dot(xb[b], wqb,
                         preferred_element_type=jnp.float32
                         ).astype(jnp.bfloat16)
            pb = jnp.zeros((SQ, D_MODEL), jnp.float32)
            for h in range(H_LOC):
                qh = qb[:, h * DH:(h + 1) * DH]
                s = lax.dot_general(
                    qh, k_all[b, h], (((1,), (1,)), ((), ())),
                    preferred_element_type=jnp.float32) * 0.125
                s = jnp.where(mask, s, jnp.float32(-1e9))
                m = jnp.max(s, axis=1, keepdims=True)
                w = jnp.exp(s - m)
                w = w / jnp.sum(w, axis=1, keepdims=True)
                ctx = jnp.dot(w.astype(jnp.bfloat16), v_all[b, h],
                              preferred_element_type=jnp.float32)
                wo_h = wo_ref[h * DH:(h + 1) * DH, :].astype(jnp.bfloat16)
                pb = pb + jnp.dot(ctx.astype(jnp.bfloat16), wo_h,
                                  preferred_element_type=jnp.float32)
            out_ref[b] = pb
            cbuf[0, b] = pb.astype(jnp.bfloat16)

        @pl.when(me == 0)
        def _():
            for n in range(N_DEV - 1):
                done = pltpu.make_async_remote_copy(
                    src_ref=kvs.at[:, :, 0:H_LOC],
                    dst_ref=kvr0,
                    send_sem=kv_send_sems.at[n],
                    recv_sem=kv_recv_sems.at[0],
                    device_id=(0,),
                    device_id_type=pl.DeviceIdType.MESH,
                )
                done.wait_send()

        @pl.when(me == 1)
        def _():
            for n in range(N_DEV - 1):
                done = pltpu.make_async_remote_copy(
                    src_ref=kvs.at[:, :, 0:H_LOC, 0:SKV1],
                    dst_ref=kvr1,
                    send_sem=kv_send_sems.at[n],
                    recv_sem=kv_recv_sems.at[1],
                    device_id=(0,),
                    device_id_type=pl.DeviceIdType.MESH,
                )
                done.wait_send()

        p1 = me ^ 1
        p2 = 3 - me
        r1 = pltpu.make_async_remote_copy(
            src_ref=cbuf.at[0], dst_ref=cbuf.at[1],
            send_sem=bf_send_sems.at[0], recv_sem=bf_recv_sems.at[0],
            device_id=(p1,), device_id_type=pl.DeviceIdType.MESH,
        )
        r1.start()
        r1.wait()
        acc = out_ref[...] + cbuf[1].astype(jnp.float32)
        out_ref[...] = acc
        cbuf[2] = acc.astype(jnp.bfloat16)
        r2 = pltpu.make_async_remote_copy(
            src_ref=cbuf.at[2], dst_ref=cbuf.at[3],
            send_sem=bf_send_sems.at[1], recv_sem=bf_recv_sems.at[1],
            device_id=(p2,), device_id_type=pl.DeviceIdType.MESH,
        )
        r2.start()
        r2.wait()
        out_ref[...] = out_ref[...] + cbuf[3].astype(jnp.float32)

    return pl.pallas_call(
        body,
        out_shape=jax.ShapeDtypeStruct((B, SQ, D_MODEL), jnp.float32),
        in_specs=[pl.BlockSpec(memory_space=pltpu.VMEM)] * 5,
        out_specs=pl.BlockSpec(memory_space=pltpu.VMEM),
        scratch_shapes=[
            pltpu.VMEM((2, B, H_TOT, SKV_SH, DH), jnp.bfloat16),
            pltpu.VMEM((2, B, H_LOC, SKV_SH, DH), jnp.bfloat16),
            pltpu.VMEM((2, B, H_LOC, SKV1, DH), jnp.bfloat16),
            pltpu.VMEM((N_DEV, B, SQ, D_MODEL), jnp.bfloat16),
            pltpu.SemaphoreType.DMA((N_DEV - 1,)),
            pltpu.SemaphoreType.DMA((2,)),
            pltpu.SemaphoreType.DMA((2,)),
            pltpu.SemaphoreType.DMA((2,)),
        ],
    )(x, Wq, K_ext, V_ext, Wo)


# device time: 47698 ns/iter; 1.3624x vs baseline; 1.2020x over previous
import jax
import jax.numpy as jnp
from jax import lax
from jax.experimental import pallas as pl
from jax.experimental.pallas import tpu as pltpu

N_DEV = 4
B, SQ, D_MODEL = 2, 256, 512
H_TOT, H_LOC, DH = 16, 4, 64
SKV_SH = 256
WIN = 128
SKV1 = 128
KV_COLS = SKV_SH + SKV1

VARIANT = "full"


def kernel(x, Wq, K_ext, V_ext, Wo):
    do_comm = VARIANT in ("full", "nocompute")
    do_compute = VARIANT in ("full", "nocomm")

    def body(x_ref, wq_ref, k_ref, v_ref, wo_ref, out_ref,
             kts, vts, ktr, vtr, cbuf, kv_send_sems, kv_recv_sems,
             bf_send_sems, bf_recv_sems):
        me = lax.axis_index("i")

        if do_comm:
            @pl.when(me < 2)
            def _():
                kts[...] = jnp.transpose(
                    k_ref[...].astype(jnp.bfloat16), (0, 2, 3, 1))

            @pl.when(me == 0)
            def _():
                ktr[:, :, :, 0:SKV_SH] = kts[:, 0:H_LOC]
                for n, j in enumerate((1, 2, 3)):
                    rdma = pltpu.make_async_remote_copy(
                        src_ref=kts.at[:, pl.ds(4 * j, H_LOC)],
                        dst_ref=ktr.at[:, :, :, 0:SKV_SH],
                        send_sem=kv_send_sems.at[n, 0],
                        recv_sem=kv_recv_sems.at[0, 0],
                        device_id=(j,),
                        device_id_type=pl.DeviceIdType.MESH,
                    )
                    rdma.start()

            @pl.when(me == 1)
            def _():
                ktr[:, :, :, SKV_SH:KV_COLS] = kts[:, H_LOC:2 * H_LOC, :, 0:SKV1]
                for n, j in enumerate((0, 2, 3)):
                    rdma = pltpu.make_async_remote_copy(
                        src_ref=kts.at[:, pl.ds(4 * j, H_LOC), :, 0:SKV1],
                        dst_ref=ktr.at[:, :, :, SKV_SH:KV_COLS],
                        send_sem=kv_send_sems.at[n, 0],
                        recv_sem=kv_recv_sems.at[1, 0],
                        device_id=(j,),
                        device_id_type=pl.DeviceIdType.MESH,
                    )
                    rdma.start()

            @pl.when(me < 2)
            def _():
                vts[...] = jnp.transpose(
                    v_ref[...].astype(jnp.bfloat16), (0, 2, 1, 3))

            @pl.when(me == 0)
            def _():
                vtr[:, :, 0:SKV_SH] = vts[:, 0:H_LOC]
                for n, j in enumerate((1, 2, 3)):
                    rdma = pltpu.make_async_remote_copy(
                        src_ref=vts.at[:, pl.ds(4 * j, H_LOC)],
                        dst_ref=vtr.at[:, :, 0:SKV_SH],
                        send_sem=kv_send_sems.at[n, 1],
                        recv_sem=kv_recv_sems.at[0, 1],
                        device_id=(j,),
                        device_id_type=pl.DeviceIdType.MESH,
                    )
                    rdma.start()

            @pl.when(me == 1)
            def _():
                vtr[:, :, SKV_SH:KV_COLS] = vts[:, H_LOC:2 * H_LOC, 0:SKV1]
                for n, j in enumerate((0, 2, 3)):
                    rdma = pltpu.make_async_remote_copy(
                        src_ref=vts.at[:, pl.ds(4 * j, H_LOC), 0:SKV1],
                        dst_ref=vtr.at[:, :, SKV_SH:KV_COLS],
                        send_sem=kv_send_sems.at[n, 1],
                        recv_sem=kv_recv_sems.at[1, 1],
                        device_id=(j,),
                        device_id_type=pl.DeviceIdType.MESH,
                    )
                    rdma.start()

        xb = x_ref[...].astype(jnp.bfloat16)
        wqb = wq_ref[...].astype(jnp.bfloat16)
        wob = wo_ref[...].astype(jnp.bfloat16)

        if do_comm:
            for src in range(2):
                @pl.when(me != src)
                def _(src=src):
                    kcols = (slice(0, SKV_SH) if src == 0
                             else slice(SKV_SH, KV_COLS))
                    ssrc = (slice(0, SKV_SH) if src == 0
                            else slice(0, SKV1))
                    krecv = pltpu.make_async_remote_copy(
                        src_ref=kts.at[:, 0:H_LOC, :, ssrc],
                        dst_ref=ktr.at[:, :, :, kcols],
                        send_sem=kv_send_sems.at[0, 0],
                        recv_sem=kv_recv_sems.at[src, 0],
                        device_id=(src,),
                        device_id_type=pl.DeviceIdType.MESH,
                    )
                    krecv.wait_recv()
                    vrecv = pltpu.make_async_remote_copy(
                        src_ref=vts.at[:, 0:H_LOC, ssrc],
                        dst_ref=vtr.at[:, :, kcols],
                        send_sem=kv_send_sems.at[0, 1],
                        recv_sem=kv_recv_sems.at[src, 1],
                        device_id=(src,),
                        device_id_type=pl.DeviceIdType.MESH,
                    )
                    vrecv.wait_recv()

        qi = lax.broadcasted_iota(jnp.int32, (SQ, KV_COLS), 0)
        ki = lax.broadcasted_iota(jnp.int32, (SQ, KV_COLS), 1)
        madd = jnp.where(jnp.abs(qi - ki) <= WIN,
                         jnp.float32(0), jnp.float32(-1e9))

        p1 = me ^ 1
        p2 = 3 - me
        ph1 = []
        ph2 = []
        for b in range(B):
            ph1.append(pltpu.make_async_remote_copy(
                src_ref=cbuf.at[0, b], dst_ref=cbuf.at[1, b],
                send_sem=bf_send_sems.at[0, b], recv_sem=bf_recv_sems.at[0, b],
                device_id=(p1,), device_id_type=pl.DeviceIdType.MESH,
            ))
            ph2.append(pltpu.make_async_remote_copy(
                src_ref=cbuf.at[2, b], dst_ref=cbuf.at[3, b],
                send_sem=bf_send_sems.at[1, b], recv_sem=bf_recv_sems.at[1, b],
                device_id=(p2,), device_id_type=pl.DeviceIdType.MESH,
            ))

        for b in range(B):
            if do_compute:
                qb = (jnp.dot(xb[b], wqb,
                              preferred_element_type=jnp.float32) * 0.125
                      ).astype(jnp.bfloat16)
                ctx_parts = []
                for h in range(H_LOC):
                    qh = qb[:, h * DH:(h + 1) * DH]
                    s = jnp.dot(qh, ktr[b, h],
                                preferred_element_type=jnp.float32) + madd
                    w = jnp.exp(s)
                    denom = jnp.sum(w, axis=1, keepdims=True)
                    ctx = jnp.dot(w.astype(jnp.bfloat16), vtr[b, h],
                                  preferred_element_type=jnp.float32)
                    ctx_parts.append((ctx / denom).astype(jnp.bfloat16))
                ctx_b = jnp.concatenate(ctx_parts, axis=1)
                pb = jnp.dot(ctx_b, wob, preferred_element_type=jnp.float32)
            else:
                pb = jnp.zeros((SQ, D_MODEL), jnp.float32)
            out_ref[b] = pb
            cbuf[0, b] = pb.astype(jnp.bfloat16)
            if do_comm:
                ph1[b].start()

        if do_comm:
            for b in range(B):
                ph1[b].wait_recv()
                acc = out_ref[b] + cbuf[1, b].astype(jnp.float32)
                out_ref[b] = acc
                cbuf[2, b] = acc.astype(jnp.bfloat16)
                ph2[b].start()
            for b in range(B):
                ph2[b].wait_recv()
                out_ref[b] = out_ref[b] + cbuf[3, b].astype(jnp.float32)

            for b in range(B):
                ph1[b].wait_send()
                ph2[b].wait_send()

            @pl.when(me == 0)
            def _():
                for n in range(N_DEV - 1):
                    kd = pltpu.make_async_remote_copy(
                        src_ref=kts.at[:, 0:H_LOC],
                        dst_ref=ktr.at[:, :, :, 0:SKV_SH],
                        send_sem=kv_send_sems.at[n, 0],
                        recv_sem=kv_recv_sems.at[0, 0],
                        device_id=(0,),
                        device_id_type=pl.DeviceIdType.MESH,
                    )
                    kd.wait_send()
                    vd = pltpu.make_async_remote_copy(
                        src_ref=vts.at[:, 0:H_LOC],
                        dst_ref=vtr.at[:, :, 0:SKV_SH],
                        send_sem=kv_send_sems.at[n, 1],
                        recv_sem=kv_recv_sems.at[0, 1],
                        device_id=(0,),
                        device_id_type=pl.DeviceIdType.MESH,
                    )
                    vd.wait_send()

            @pl.when(me == 1)
            def _():
                for n in range(N_DEV - 1):
                    kd = pltpu.make_async_remote_copy(
                        src_ref=kts.at[:, 0:H_LOC, :, 0:SKV1],
                        dst_ref=ktr.at[:, :, :, SKV_SH:KV_COLS],
                        send_sem=kv_send_sems.at[n, 0],
                        recv_sem=kv_recv_sems.at[1, 0],
                        device_id=(0,),
                        device_id_type=pl.DeviceIdType.MESH,
                    )
                    kd.wait_send()
                    vd = pltpu.make_async_remote_copy(
                        src_ref=vts.at[:, 0:H_LOC, 0:SKV1],
                        dst_ref=vtr.at[:, :, SKV_SH:KV_COLS],
                        send_sem=kv_send_sems.at[n, 1],
                        recv_sem=kv_recv_sems.at[1, 1],
                        device_id=(0,),
                        device_id_type=pl.DeviceIdType.MESH,
                    )
                    vd.wait_send()

    return pl.pallas_call(
        body,
        out_shape=jax.ShapeDtypeStruct((B, SQ, D_MODEL), jnp.float32),
        in_specs=[pl.BlockSpec(memory_space=pltpu.VMEM)] * 5,
        out_specs=pl.BlockSpec(memory_space=pltpu.VMEM),
        scratch_shapes=[
            pltpu.VMEM((B, H_TOT, DH, SKV_SH), jnp.bfloat16),
            pltpu.VMEM((B, H_TOT, SKV_SH, DH), jnp.bfloat16),
            pltpu.VMEM((B, H_LOC, DH, KV_COLS), jnp.bfloat16),
            pltpu.VMEM((B, H_LOC, KV_COLS, DH), jnp.bfloat16),
            pltpu.VMEM((4, B, SQ, D_MODEL), jnp.bfloat16),
            pltpu.SemaphoreType.DMA((N_DEV - 1, 2)),
            pltpu.SemaphoreType.DMA((2, 2)),
            pltpu.SemaphoreType.DMA((2, B)),
            pltpu.SemaphoreType.DMA((2, B)),
        ],
    )(x, Wq, K_ext, V_ext, Wo)


# device time: 47491 ns/iter; 1.3683x vs baseline; 1.0044x over previous
import jax
import jax.numpy as jnp
from jax import lax
from jax.experimental import pallas as pl
from jax.experimental.pallas import tpu as pltpu

N_DEV = 4
B, SQ, D_MODEL = 2, 256, 512
H_TOT, H_LOC, DH = 16, 4, 64
SKV_SH = 256
WIN = 128
SKV1 = 128
KV_COLS = SKV_SH + SKV1

VARIANT = "full"


def kernel(x, Wq, K_ext, V_ext, Wo):
    do_comm = VARIANT in ("full", "nocompute")
    do_compute = VARIANT in ("full", "nocomm")

    def body(x_ref, wq_ref, k_ref, v_ref, wo_ref, out_ref,
             kraw, vraw, kts, vts, ktr, vtr, cbuf,
             copy_sems, kv_send_sems, kv_recv_sems,
             bf_send_sems, bf_recv_sems):
        me = lax.axis_index("i")

        if do_comm:
            @pl.when(me < 2)
            def _():
                kcp = pltpu.make_async_copy(k_ref, kraw, copy_sems.at[0])
                kcp.start()
                vcp = pltpu.make_async_copy(v_ref, vraw, copy_sems.at[1])
                vcp.start()
                kcp.wait()
                kts[...] = jnp.transpose(
                    kraw[...].astype(jnp.bfloat16), (0, 2, 3, 1))

            @pl.when(me == 0)
            def _():
                ktr[:, :, :, 0:SKV_SH] = kts[:, 0:H_LOC]
                for n, j in enumerate((1, 2, 3)):
                    rdma = pltpu.make_async_remote_copy(
                        src_ref=kts.at[:, pl.ds(4 * j, H_LOC)],
                        dst_ref=ktr.at[:, :, :, 0:SKV_SH],
                        send_sem=kv_send_sems.at[n, 0],
                        recv_sem=kv_recv_sems.at[0, 0],
                        device_id=(j,),
                        device_id_type=pl.DeviceIdType.MESH,
                    )
                    rdma.start()

            @pl.when(me == 1)
            def _():
                ktr[:, :, :, SKV_SH:KV_COLS] = kts[:, H_LOC:2 * H_LOC, :, 0:SKV1]
                for n, j in enumerate((0, 2, 3)):
                    rdma = pltpu.make_async_remote_copy(
                        src_ref=kts.at[:, pl.ds(4 * j, H_LOC), :, 0:SKV1],
                        dst_ref=ktr.at[:, :, :, SKV_SH:KV_COLS],
                        send_sem=kv_send_sems.at[n, 0],
                        recv_sem=kv_recv_sems.at[1, 0],
                        device_id=(j,),
                        device_id_type=pl.DeviceIdType.MESH,
                    )
                    rdma.start()

            @pl.when(me < 2)
            def _():
                vcp = pltpu.make_async_copy(v_ref, vraw, copy_sems.at[1])
                vcp.wait()
                vts[...] = jnp.transpose(
                    vraw[...].astype(jnp.bfloat16), (0, 2, 1, 3))

            @pl.when(me == 0)
            def _():
                vtr[:, :, 0:SKV_SH] = vts[:, 0:H_LOC]
                for n, j in enumerate((1, 2, 3)):
                    rdma = pltpu.make_async_remote_copy(
                        src_ref=vts.at[:, pl.ds(4 * j, H_LOC)],
                        dst_ref=vtr.at[:, :, 0:SKV_SH],
                        send_sem=kv_send_sems.at[n, 1],
                        recv_sem=kv_recv_sems.at[0, 1],
                        device_id=(j,),
                        device_id_type=pl.DeviceIdType.MESH,
                    )
                    rdma.start()

            @pl.when(me == 1)
            def _():
                vtr[:, :, SKV_SH:KV_COLS] = vts[:, H_LOC:2 * H_LOC, 0:SKV1]
                for n, j in enumerate((0, 2, 3)):
                    rdma = pltpu.make_async_remote_copy(
                        src_ref=vts.at[:, pl.ds(4 * j, H_LOC), 0:SKV1],
                        dst_ref=vtr.at[:, :, SKV_SH:KV_COLS],
                        send_sem=kv_send_sems.at[n, 1],
                        recv_sem=kv_recv_sems.at[1, 1],
                        device_id=(j,),
                        device_id_type=pl.DeviceIdType.MESH,
                    )
                    rdma.start()

        xb = x_ref[...].astype(jnp.bfloat16)
        wqb = wq_ref[...].astype(jnp.bfloat16)
        wob = wo_ref[...].astype(jnp.bfloat16)

        def wait_kv(src, tensor):
            kcols = slice(0, SKV_SH) if src == 0 else slice(SKV_SH, KV_COLS)
            ssrc = slice(0, SKV_SH) if src == 0 else slice(0, SKV1)
            if tensor == 0:
                rdma = pltpu.make_async_remote_copy(
                    src_ref=kts.at[:, 0:H_LOC, :, ssrc],
                    dst_ref=ktr.at[:, :, :, kcols],
                    send_sem=kv_send_sems.at[0, 0],
                    recv_sem=kv_recv_sems.at[src, 0],
                    device_id=(src,),
                    device_id_type=pl.DeviceIdType.MESH,
                )
            else:
                rdma = pltpu.make_async_remote_copy(
                    src_ref=vts.at[:, 0:H_LOC, ssrc],
                    dst_ref=vtr.at[:, :, kcols],
                    send_sem=kv_send_sems.at[0, 1],
                    recv_sem=kv_recv_sems.at[src, 1],
                    device_id=(src,),
                    device_id_type=pl.DeviceIdType.MESH,
                )
            rdma.wait_recv()

        if do_comm:
            for src in range(2):
                @pl.when(me != src)
                def _(src=src):
                    wait_kv(src, 0)

        qi = lax.broadcasted_iota(jnp.int32, (SQ, KV_COLS), 0)
        ki = lax.broadcasted_iota(jnp.int32, (SQ, KV_COLS), 1)
        madd = jnp.where(jnp.abs(qi - ki) <= WIN,
                         jnp.float32(0), jnp.float32(-1e9))

        w_all = []
        denom_all = []
        q_all = []
        if do_compute:
            for b in range(B):
                qb = (jnp.dot(xb[b], wqb,
                              preferred_element_type=jnp.float32) * 0.125
                      ).astype(jnp.bfloat16)
                q_all.append(qb)
                for h in range(H_LOC):
                    qh = qb[:, h * DH:(h + 1) * DH]
                    s = jnp.dot(qh, ktr[b, h],
                                preferred_element_type=jnp.float32) + madd
                    w = jnp.exp(s)
                    denom_all.append(jnp.sum(w, axis=1, keepdims=True))
                    w_all.append(w.astype(jnp.bfloat16))

        if do_comm:
            for src in range(2):
                @pl.when(me != src)
                def _(src=src):
                    wait_kv(src, 1)

        p1 = me ^ 1
        p2 = 3 - me
        ph1 = []
        ph2 = []
        for b in range(B):
            ph1.append(pltpu.make_async_remote_copy(
                src_ref=cbuf.at[0, b], dst_ref=cbuf.at[1, b],
                send_sem=bf_send_sems.at[0, b], recv_sem=bf_recv_sems.at[0, b],
                device_id=(p1,), device_id_type=pl.DeviceIdType.MESH,
            ))
            ph2.append(pltpu.make_async_remote_copy(
                src_ref=cbuf.at[2, b], dst_ref=cbuf.at[3, b],
                send_sem=bf_send_sems.at[1, b], recv_sem=bf_recv_sems.at[1, b],
                device_id=(p2,), device_id_type=pl.DeviceIdType.MESH,
            ))

        pb_all = []
        for b in range(B):
            if do_compute:
                ctx_parts = []
                for h in range(H_LOC):
                    i = b * H_LOC + h
                    ctx = jnp.dot(w_all[i], vtr[b, h],
                                  preferred_element_type=jnp.float32)
                    ctx_parts.append((ctx / denom_all[i]).astype(jnp.bfloat16))
                ctx_b = jnp.concatenate(ctx_parts, axis=1)
                pb = jnp.dot(ctx_b, wob, preferred_element_type=jnp.float32)
            else:
                pb = jnp.zeros((SQ, D_MODEL), jnp.float32)
            pb_all.append(pb)
            cbuf[0, b] = pb.astype(jnp.bfloat16)
            if do_comm:
                ph1[b].start()

        if do_comm:
            acc_all = []
            for b in range(B):
                ph1[b].wait_recv()
                acc = pb_all[b] + cbuf[1, b].astype(jnp.float32)
                acc_all.append(acc)
                cbuf[2, b] = acc.astype(jnp.bfloat16)
                ph2[b].start()
            for b in range(B):
                ph2[b].wait_recv()
                out_ref[b] = acc_all[b] + cbuf[3, b].astype(jnp.float32)

            for b in range(B):
                ph1[b].wait_send()
                ph2[b].wait_send()

            def retire_sends(src):
                ssrc = slice(0, SKV_SH) if src == 0 else slice(0, SKV1)
                kcols = slice(0, SKV_SH) if src == 0 else slice(SKV_SH, KV_COLS)
                for n in range(N_DEV - 1):
                    kd = pltpu.make_async_remote_copy(
                        src_ref=kts.at[:, 0:H_LOC, :, ssrc],
                        dst_ref=ktr.at[:, :, :, kcols],
                        send_sem=kv_send_sems.at[n, 0],
                        recv_sem=kv_recv_sems.at[src, 0],
                        device_id=(0,),
                        device_id_type=pl.DeviceIdType.MESH,
                    )
                    kd.wait_send()
                    vd = pltpu.make_async_remote_copy(
                        src_ref=vts.at[:, 0:H_LOC, ssrc],
                        dst_ref=vtr.at[:, :, kcols],
                        send_sem=kv_send_sems.at[n, 1],
                        recv_sem=kv_recv_sems.at[src, 1],
                        device_id=(0,),
                        device_id_type=pl.DeviceIdType.MESH,
                    )
                    vd.wait_send()

            @pl.when(me == 0)
            def _():
                retire_sends(0)

            @pl.when(me == 1)
            def _():
                retire_sends(1)
        else:
            for b in range(B):
                out_ref[b] = pb_all[b]

    return pl.pallas_call(
        body,
        out_shape=jax.ShapeDtypeStruct((B, SQ, D_MODEL), jnp.float32),
        in_specs=[
            pl.BlockSpec(memory_space=pltpu.VMEM),
            pl.BlockSpec(memory_space=pltpu.VMEM),
            pl.BlockSpec(memory_space=pltpu.MemorySpace.HBM),
            pl.BlockSpec(memory_space=pltpu.MemorySpace.HBM),
            pl.BlockSpec(memory_space=pltpu.VMEM),
        ],
        out_specs=pl.BlockSpec(memory_space=pltpu.VMEM),
        scratch_shapes=[
            pltpu.VMEM((B, SKV_SH, H_TOT, DH), jnp.float32),
            pltpu.VMEM((B, SKV_SH, H_TOT, DH), jnp.float32),
            pltpu.VMEM((B, H_TOT, DH, SKV_SH), jnp.bfloat16),
            pltpu.VMEM((B, H_TOT, SKV_SH, DH), jnp.bfloat16),
            pltpu.VMEM((B, H_LOC, DH, KV_COLS), jnp.bfloat16),
            pltpu.VMEM((B, H_LOC, KV_COLS, DH), jnp.bfloat16),
            pltpu.VMEM((4, B, SQ, D_MODEL), jnp.bfloat16),
            pltpu.SemaphoreType.DMA((2,)),
            pltpu.SemaphoreType.DMA((N_DEV - 1, 2)),
            pltpu.SemaphoreType.DMA((2, 2)),
            pltpu.SemaphoreType.DMA((2, B)),
            pltpu.SemaphoreType.DMA((2, B)),
        ],
    )(x, Wq, K_ext, V_ext, Wo)


# device time: 46292 ns/iter; 1.4038x vs baseline; 1.0259x over previous
import jax
import jax.numpy as jnp
from jax import lax
from jax.experimental import pallas as pl
from jax.experimental.pallas import tpu as pltpu

N_DEV = 4
B, SQ, D_MODEL = 2, 256, 512
H_TOT, H_LOC, DH = 16, 4, 64
SKV_SH = 256
WIN = 128
SKV1 = 128
KV_COLS = SKV_SH + SKV1

VARIANT = "full"


def kernel(x, Wq, K_ext, V_ext, Wo):
    do_comm = VARIANT in ("full", "nocompute")
    do_compute = VARIANT in ("full", "nocomm")

    def body(x_ref, wq_ref, k_ref, v_ref, wo_ref, out_ref,
             kraw, vraw, kts, vts, ktr, vtr, cbuf,
             copy_sems, kv_send_sems, kv_recv_sems,
             bf_send_sems, bf_recv_sems):
        me = lax.axis_index("i")

        if do_comm:
            @pl.when(me < 2)
            def _():
                kcp = pltpu.make_async_copy(k_ref, kraw, copy_sems.at[0])
                kcp.start()
                vcp = pltpu.make_async_copy(v_ref, vraw, copy_sems.at[1])
                vcp.start()
                kcp.wait()
                kts[...] = jnp.transpose(
                    kraw[...].astype(jnp.bfloat16), (0, 2, 3, 1))
                vcp.wait()
                vts[...] = jnp.transpose(
                    vraw[...].astype(jnp.bfloat16), (0, 2, 1, 3))

            @pl.when(me == 0)
            def _():
                ktr[:, :, :, 0:SKV_SH] = kts[:, 0:H_LOC]
                vtr[:, :, 0:SKV_SH] = vts[:, 0:H_LOC]

            @pl.when(me == 1)
            def _():
                ktr[:, :, :, SKV_SH:KV_COLS] = kts[:, H_LOC:2 * H_LOC, :, 0:SKV1]
                vtr[:, :, SKV_SH:KV_COLS] = vts[:, H_LOC:2 * H_LOC, 0:SKV1]

        xb = x_ref[...].astype(jnp.bfloat16)
        wqb = wq_ref[...].astype(jnp.bfloat16)
        wob = wo_ref[...].astype(jnp.bfloat16)
        q_all = []
        if do_compute:
            for b in range(B):
                q_all.append(
                    (jnp.dot(xb[b], wqb,
                             preferred_element_type=jnp.float32) * 0.125
                     ).astype(jnp.bfloat16))

        qi = lax.broadcasted_iota(jnp.int32, (SQ, KV_COLS), 0)
        ki = lax.broadcasted_iota(jnp.int32, (SQ, KV_COLS), 1)
        madd = jnp.where(jnp.abs(qi - ki) <= WIN,
                         jnp.float32(0), jnp.float32(-1e9))

        if do_comm:
            barrier_sem = pltpu.get_barrier_semaphore()
            for d in range(1, N_DEV):
                pl.semaphore_signal(
                    barrier_sem, inc=1,
                    device_id=((me + d) % N_DEV,),
                    device_id_type=pl.DeviceIdType.MESH,
                )
            pl.semaphore_wait(barrier_sem, N_DEV - 1)

            @pl.when(me == 0)
            def _():
                for n, j in enumerate((1, 2, 3)):
                    pltpu.make_async_remote_copy(
                        src_ref=kts.at[:, pl.ds(4 * j, H_LOC)],
                        dst_ref=ktr.at[:, :, :, 0:SKV_SH],
                        send_sem=kv_send_sems.at[n, 0],
                        recv_sem=kv_recv_sems.at[0, 0],
                        device_id=(j,),
                        device_id_type=pl.DeviceIdType.MESH,
                    ).start()
                for n, j in enumerate((1, 2, 3)):
                    pltpu.make_async_remote_copy(
                        src_ref=vts.at[:, pl.ds(4 * j, H_LOC)],
                        dst_ref=vtr.at[:, :, 0:SKV_SH],
                        send_sem=kv_send_sems.at[n, 1],
                        recv_sem=kv_recv_sems.at[0, 1],
                        device_id=(j,),
                        device_id_type=pl.DeviceIdType.MESH,
                    ).start()

            @pl.when(me == 1)
            def _():
                for n, j in enumerate((0, 2, 3)):
                    pltpu.make_async_remote_copy(
                        src_ref=kts.at[:, pl.ds(4 * j, H_LOC), :, 0:SKV1],
                        dst_ref=ktr.at[:, :, :, SKV_SH:KV_COLS],
                        send_sem=kv_send_sems.at[n, 0],
                        recv_sem=kv_recv_sems.at[1, 0],
                        device_id=(j,),
                        device_id_type=pl.DeviceIdType.MESH,
                    ).start()
                for n, j in enumerate((0, 2, 3)):
                    pltpu.make_async_remote_copy(
                        src_ref=vts.at[:, pl.ds(4 * j, H_LOC), 0:SKV1],
                        dst_ref=vtr.at[:, :, SKV_SH:KV_COLS],
                        send_sem=kv_send_sems.at[n, 1],
                        recv_sem=kv_recv_sems.at[1, 1],
                        device_id=(j,),
                        device_id_type=pl.DeviceIdType.MESH,
                    ).start()

        def wait_kv(src, tensor):
            kcols = slice(0, SKV_SH) if src == 0 else slice(SKV_SH, KV_COLS)
            ssrc = slice(0, SKV_SH) if src == 0 else slice(0, SKV1)
            if tensor == 0:
                rdma = pltpu.make_async_remote_copy(
                    src_ref=kts.at[:, 0:H_LOC, :, ssrc],
                    dst_ref=ktr.at[:, :, :, kcols],
                    send_sem=kv_send_sems.at[0, 0],
                    recv_sem=kv_recv_sems.at[src, 0],
                    device_id=(src,),
                    device_id_type=pl.DeviceIdType.MESH,
                )
            else:
                rdma = pltpu.make_async_remote_copy(
                    src_ref=vts.at[:, 0:H_LOC, ssrc],
                    dst_ref=vtr.at[:, :, kcols],
                    send_sem=kv_send_sems.at[0, 1],
                    recv_sem=kv_recv_sems.at[src, 1],
                    device_id=(src,),
                    device_id_type=pl.DeviceIdType.MESH,
                )
            rdma.wait_recv()

        if do_comm:
            for src in range(2):
                @pl.when(me != src)
                def _(src=src):
                    wait_kv(src, 0)

        w_all = []
        denom_all = []
        if do_compute:
            for b in range(B):
                for h in range(H_LOC):
                    qh = q_all[b][:, h * DH:(h + 1) * DH]
                    s = jnp.dot(qh, ktr[b, h],
                                preferred_element_type=jnp.float32) + madd
                    w = jnp.exp(s)
                    denom_all.append(jnp.sum(w, axis=1, keepdims=True))
                    w_all.append(w.astype(jnp.bfloat16))

        if do_comm:
            for src in range(2):
                @pl.when(me != src)
                def _(src=src):
                    wait_kv(src, 1)

        p1 = me ^ 1
        p2 = 3 - me
        ph1 = []
        ph2 = []
        for b in range(B):
            ph1.append(pltpu.make_async_remote_copy(
                src_ref=cbuf.at[0, b], dst_ref=cbuf.at[1, b],
                send_sem=bf_send_sems.at[0, b], recv_sem=bf_recv_sems.at[0, b],
                device_id=(p1,), device_id_type=pl.DeviceIdType.MESH,
            ))
            ph2.append(pltpu.make_async_remote_copy(
                src_ref=cbuf.at[2, b], dst_ref=cbuf.at[3, b],
                send_sem=bf_send_sems.at[1, b], recv_sem=bf_recv_sems.at[1, b],
                device_id=(p2,), device_id_type=pl.DeviceIdType.MESH,
            ))

        pb_all = []
        for b in range(B):
            if do_compute:
                ctx_parts = []
                for h in range(H_LOC):
                    i = b * H_LOC + h
                    ctx = jnp.dot(w_all[i], vtr[b, h],
                                  preferred_element_type=jnp.float32)
                    ctx_parts.append((ctx / denom_all[i]).astype(jnp.bfloat16))
                ctx_b = jnp.concatenate(ctx_parts, axis=1)
                pb = jnp.dot(ctx_b, wob, preferred_element_type=jnp.float32)
            else:
                pb = jnp.zeros((SQ, D_MODEL), jnp.float32)
            pb_all.append(pb)
            cbuf[0, b] = pb.astype(jnp.bfloat16)
            if do_comm:
                ph1[b].start()

        if do_comm:
            acc_all = []
            for b in range(B):
                ph1[b].wait_recv()
                acc = pb_all[b] + cbuf[1, b].astype(jnp.float32)
                acc_all.append(acc)
                cbuf[2, b] = acc.astype(jnp.bfloat16)
                ph2[b].start()
            for b in range(B):
                ph2[b].wait_recv()
                out_ref[b] = acc_all[b] + cbuf[3, b].astype(jnp.float32)

            for b in range(B):
                ph1[b].wait_send()
                ph2[b].wait_send()

            def retire_sends(src):
                ssrc = slice(0, SKV_SH) if src == 0 else slice(0, SKV1)
                kcols = slice(0, SKV_SH) if src == 0 else slice(SKV_SH, KV_COLS)
                for n in range(N_DEV - 1):
                    pltpu.make_async_remote_copy(
                        src_ref=kts.at[:, 0:H_LOC, :, ssrc],
                        dst_ref=ktr.at[:, :, :, kcols],
                        send_sem=kv_send_sems.at[n, 0],
                        recv_sem=kv_recv_sems.at[src, 0],
                        device_id=(0,),
                        device_id_type=pl.DeviceIdType.MESH,
                    ).wait_send()
                    pltpu.make_async_remote_copy(
                        src_ref=vts.at[:, 0:H_LOC, ssrc],
                        dst_ref=vtr.at[:, :, kcols],
                        send_sem=kv_send_sems.at[n, 1],
                        recv_sem=kv_recv_sems.at[src, 1],
                        device_id=(0,),
                        device_id_type=pl.DeviceIdType.MESH,
                    ).wait_send()

            @pl.when(me == 0)
            def _():
                retire_sends(0)

            @pl.when(me == 1)
            def _():
                retire_sends(1)
        else:
            for b in range(B):
                out_ref[b] = pb_all[b]

    return pl.pallas_call(
        body,
        out_shape=jax.ShapeDtypeStruct((B, SQ, D_MODEL), jnp.float32),
        in_specs=[
            pl.BlockSpec(memory_space=pltpu.VMEM),
            pl.BlockSpec(memory_space=pltpu.VMEM),
            pl.BlockSpec(memory_space=pltpu.MemorySpace.HBM),
            pl.BlockSpec(memory_space=pltpu.MemorySpace.HBM),
            pl.BlockSpec(memory_space=pltpu.VMEM),
        ],
        out_specs=pl.BlockSpec(memory_space=pltpu.VMEM),
        scratch_shapes=[
            pltpu.VMEM((B, SKV_SH, H_TOT, DH), jnp.float32),
            pltpu.VMEM((B, SKV_SH, H_TOT, DH), jnp.float32),
            pltpu.VMEM((B, H_TOT, DH, SKV_SH), jnp.bfloat16),
            pltpu.VMEM((B, H_TOT, SKV_SH, DH), jnp.bfloat16),
            pltpu.VMEM((B, H_LOC, DH, KV_COLS), jnp.bfloat16),
            pltpu.VMEM((B, H_LOC, KV_COLS, DH), jnp.bfloat16),
            pltpu.VMEM((4, B, SQ, D_MODEL), jnp.bfloat16),
            pltpu.SemaphoreType.DMA((2,)),
            pltpu.SemaphoreType.DMA((N_DEV - 1, 2)),
            pltpu.SemaphoreType.DMA((2, 2)),
            pltpu.SemaphoreType.DMA((2, B)),
            pltpu.SemaphoreType.DMA((2, B)),
        ],
        compiler_params=pltpu.CompilerParams(collective_id=0),
    )(x, Wq, K_ext, V_ext, Wo)


# device time: 44941 ns/iter; 1.4460x vs baseline; 1.0301x over previous
import jax
import jax.numpy as jnp
from jax import lax
from jax.experimental import pallas as pl
from jax.experimental.pallas import tpu as pltpu

N_DEV = 4
B, SQ, D_MODEL = 2, 256, 512
H_TOT, H_LOC, DH = 16, 4, 64
SKV_SH = 256
WIN = 128
SKV1 = 128
KV_COLS = SKV_SH + SKV1

VARIANT = "full"


def kernel(x, Wq, K_ext, V_ext, Wo):
    do_comm = VARIANT in ("full", "nocompute")
    do_compute = VARIANT in ("full", "nocomm")

    def body(x_ref, wq_ref, k_ref, v_ref, wo_ref, out_ref,
             kraw, vraw, kts, vts, ktr, vtr, cbuf,
             copy_sems, kv_send_sems, kv_recv_sems,
             bf_send_sems, bf_recv_sems):
        me = lax.axis_index("i")

        if do_comm:
            @pl.when(me < 2)
            def _():
                kcp = pltpu.make_async_copy(k_ref, kraw, copy_sems.at[0])
                kcp.start()
                vcp = pltpu.make_async_copy(v_ref, vraw, copy_sems.at[1])
                vcp.start()
                kcp.wait()
                kts[...] = jnp.transpose(
                    kraw[...].astype(jnp.bfloat16), (0, 2, 3, 1))
                vcp.wait()
                vts[...] = jnp.transpose(
                    vraw[...].astype(jnp.bfloat16), (0, 2, 1, 3))

            @pl.when(me == 0)
            def _():
                ktr[:, :, :, 0:SKV_SH] = kts[:, 0:H_LOC]
                vtr[:, :, 0:SKV_SH] = vts[:, 0:H_LOC]

            @pl.when(me == 1)
            def _():
                ktr[:, :, :, SKV_SH:KV_COLS] = kts[:, H_LOC:2 * H_LOC, :, 0:SKV1]
                vtr[:, :, SKV_SH:KV_COLS] = vts[:, H_LOC:2 * H_LOC, 0:SKV1]

        xb = x_ref[...].astype(jnp.bfloat16)
        wqb = wq_ref[...].astype(jnp.bfloat16)
        wob = wo_ref[...].astype(jnp.bfloat16)
        q_all = []
        if do_compute:
            for b in range(B):
                q_all.append(
                    (jnp.dot(xb[b], wqb,
                             preferred_element_type=jnp.float32) * 0.125
                     ).astype(jnp.bfloat16))

        qi = lax.broadcasted_iota(jnp.int32, (SQ, KV_COLS), 0)
        ki = lax.broadcasted_iota(jnp.int32, (SQ, KV_COLS), 1)
        madd = jnp.where(jnp.abs(qi - ki) <= WIN,
                         jnp.float32(0), jnp.float32(-1e9))

        if do_comm or VARIANT == "barrieronly":
            barrier_sem = pltpu.get_barrier_semaphore()
            for d in range(1, N_DEV):
                pl.semaphore_signal(
                    barrier_sem, inc=1,
                    device_id=((me + d) % N_DEV,),
                    device_id_type=pl.DeviceIdType.MESH,
                )
            pl.semaphore_wait(barrier_sem, N_DEV - 1)

            @pl.when(me == 0)
            def _():
                for n, j in enumerate((1, 2, 3)):
                    pltpu.make_async_remote_copy(
                        src_ref=kts.at[:, pl.ds(4 * j, H_LOC)],
                        dst_ref=ktr.at[:, :, :, 0:SKV_SH],
                        send_sem=kv_send_sems.at[n, 0],
                        recv_sem=kv_recv_sems.at[0, 0],
                        device_id=(j,),
                        device_id_type=pl.DeviceIdType.MESH,
                    ).start()
                for n, j in enumerate((1, 2, 3)):
                    pltpu.make_async_remote_copy(
                        src_ref=vts.at[:, pl.ds(4 * j, H_LOC)],
                        dst_ref=vtr.at[:, :, 0:SKV_SH],
                        send_sem=kv_send_sems.at[n, 1],
                        recv_sem=kv_recv_sems.at[0, 1],
                        device_id=(j,),
                        device_id_type=pl.DeviceIdType.MESH,
                    ).start()

            @pl.when(me == 1)
            def _():
                for n, j in enumerate((0, 2, 3)):
                    pltpu.make_async_remote_copy(
                        src_ref=kts.at[:, pl.ds(4 * j, H_LOC), :, 0:SKV1],
                        dst_ref=ktr.at[:, :, :, SKV_SH:KV_COLS],
                        send_sem=kv_send_sems.at[n, 0],
                        recv_sem=kv_recv_sems.at[1, 0],
                        device_id=(j,),
                        device_id_type=pl.DeviceIdType.MESH,
                    ).start()
                for n, j in enumerate((0, 2, 3)):
                    pltpu.make_async_remote_copy(
                        src_ref=vts.at[:, pl.ds(4 * j, H_LOC), 0:SKV1],
                        dst_ref=vtr.at[:, :, SKV_SH:KV_COLS],
                        send_sem=kv_send_sems.at[n, 1],
                        recv_sem=kv_recv_sems.at[1, 1],
                        device_id=(j,),
                        device_id_type=pl.DeviceIdType.MESH,
                    ).start()

        def wait_kv(src, tensor):
            kcols = slice(0, SKV_SH) if src == 0 else slice(SKV_SH, KV_COLS)
            ssrc = slice(0, SKV_SH) if src == 0 else slice(0, SKV1)
            if tensor == 0:
                rdma = pltpu.make_async_remote_copy(
                    src_ref=kts.at[:, 0:H_LOC, :, ssrc],
                    dst_ref=ktr.at[:, :, :, kcols],
                    send_sem=kv_send_sems.at[0, 0],
                    recv_sem=kv_recv_sems.at[src, 0],
                    device_id=(src,),
                    device_id_type=pl.DeviceIdType.MESH,
                )
            else:
                rdma = pltpu.make_async_remote_copy(
                    src_ref=vts.at[:, 0:H_LOC, ssrc],
                    dst_ref=vtr.at[:, :, kcols],
                    send_sem=kv_send_sems.at[0, 1],
                    recv_sem=kv_recv_sems.at[src, 1],
                    device_id=(src,),
                    device_id_type=pl.DeviceIdType.MESH,
                )
            rdma.wait_recv()

        if do_comm:
            for src in range(2):
                @pl.when(me != src)
                def _(src=src):
                    wait_kv(src, 0)

        p1 = me ^ 1
        p2 = 3 - me
        HALF = SQ // 2
        CHUNKS = [(b, off) for b in range(B) for off in (0, HALF)]
        ph1 = {}
        ph2 = {}
        for c, (b, off) in enumerate(CHUNKS):
            ph1[c] = pltpu.make_async_remote_copy(
                src_ref=cbuf.at[0, b, pl.ds(off, HALF)],
                dst_ref=cbuf.at[1, b, pl.ds(off, HALF)],
                send_sem=bf_send_sems.at[0, c], recv_sem=bf_recv_sems.at[0, c],
                device_id=(p1,), device_id_type=pl.DeviceIdType.MESH,
            )
            ph2[c] = pltpu.make_async_remote_copy(
                src_ref=cbuf.at[2, b, pl.ds(off, HALF)],
                dst_ref=cbuf.at[3, b, pl.ds(off, HALF)],
                send_sem=bf_send_sems.at[1, c], recv_sem=bf_recv_sems.at[1, c],
                device_id=(p2,), device_id_type=pl.DeviceIdType.MESH,
            )

        pb_all = []
        for b in range(B):
            if do_compute:
                w_b = []
                denom_b = []
                for h in range(H_LOC):
                    qh = q_all[b][:, h * DH:(h + 1) * DH]
                    s = jnp.dot(qh, ktr[b, h],
                                preferred_element_type=jnp.float32) + madd
                    w = jnp.exp(s)
                    denom_b.append(jnp.sum(w, axis=1, keepdims=True))
                    w_b.append(w.astype(jnp.bfloat16))
            if do_comm and b == 0:
                for src in range(2):
                    @pl.when(me != src)
                    def _(src=src):
                        wait_kv(src, 1)
            if do_compute:
                ctx_parts = []
                for h in range(H_LOC):
                    ctx = jnp.dot(w_b[h], vtr[b, h],
                                  preferred_element_type=jnp.float32)
                    ctx_parts.append((ctx / denom_b[h]).astype(jnp.bfloat16))
                ctx_b = jnp.concatenate(ctx_parts, axis=1)
                pb = jnp.dot(ctx_b, wob, preferred_element_type=jnp.float32)
            else:
                pb = jnp.zeros((SQ, D_MODEL), jnp.float32)
            pb_all.append(pb)
            cbuf[0, b] = pb.astype(jnp.bfloat16)
            if do_comm:
                ph1[2 * b].start()
                ph1[2 * b + 1].start()

        if do_comm:
            acc_all = {}
            for c, (b, off) in enumerate(CHUNKS):
                ph1[c].wait_recv()
                acc = (pb_all[b][off:off + HALF]
                       + cbuf[1, b, off:off + HALF].astype(jnp.float32))
                acc_all[c] = acc
                cbuf[2, b, off:off + HALF] = acc.astype(jnp.bfloat16)
                ph2[c].start()
            for c, (b, off) in enumerate(CHUNKS):
                ph2[c].wait_recv()
                out_ref[b, off:off + HALF] = (
                    acc_all[c] + cbuf[3, b, off:off + HALF].astype(jnp.float32))

            for c in range(len(CHUNKS)):
                ph1[c].wait_send()
                ph2[c].wait_send()

            def retire_sends(src):
                ssrc = slice(0, SKV_SH) if src == 0 else slice(0, SKV1)
                kcols = slice(0, SKV_SH) if src == 0 else slice(SKV_SH, KV_COLS)
                for n in range(N_DEV - 1):
                    pltpu.make_async_remote_copy(
                        src_ref=kts.at[:, 0:H_LOC, :, ssrc],
                        dst_ref=ktr.at[:, :, :, kcols],
                        send_sem=kv_send_sems.at[n, 0],
                        recv_sem=kv_recv_sems.at[src, 0],
                        device_id=(0,),
                        device_id_type=pl.DeviceIdType.MESH,
                    ).wait_send()
                    pltpu.make_async_remote_copy(
                        src_ref=vts.at[:, 0:H_LOC, ssrc],
                        dst_ref=vtr.at[:, :, kcols],
                        send_sem=kv_send_sems.at[n, 1],
                        recv_sem=kv_recv_sems.at[src, 1],
                        device_id=(0,),
                        device_id_type=pl.DeviceIdType.MESH,
                    ).wait_send()

            @pl.when(me == 0)
            def _():
                retire_sends(0)

            @pl.when(me == 1)
            def _():
                retire_sends(1)
        else:
            for b in range(B):
                out_ref[b] = pb_all[b]

    return pl.pallas_call(
        body,
        out_shape=jax.ShapeDtypeStruct((B, SQ, D_MODEL), jnp.float32),
        in_specs=[
            pl.BlockSpec(memory_space=pltpu.VMEM),
            pl.BlockSpec(memory_space=pltpu.VMEM),
            pl.BlockSpec(memory_space=pltpu.MemorySpace.HBM),
            pl.BlockSpec(memory_space=pltpu.MemorySpace.HBM),
            pl.BlockSpec(memory_space=pltpu.VMEM),
        ],
        out_specs=pl.BlockSpec(memory_space=pltpu.VMEM),
        scratch_shapes=[
            pltpu.VMEM((B, SKV_SH, H_TOT, DH), jnp.float32),
            pltpu.VMEM((B, SKV_SH, H_TOT, DH), jnp.float32),
            pltpu.VMEM((B, H_TOT, DH, SKV_SH), jnp.bfloat16),
            pltpu.VMEM((B, H_TOT, SKV_SH, DH), jnp.bfloat16),
            pltpu.VMEM((B, H_LOC, DH, KV_COLS), jnp.bfloat16),
            pltpu.VMEM((B, H_LOC, KV_COLS, DH), jnp.bfloat16),
            pltpu.VMEM((4, B, SQ, D_MODEL), jnp.bfloat16),
            pltpu.SemaphoreType.DMA((2,)),
            pltpu.SemaphoreType.DMA((N_DEV - 1, 2)),
            pltpu.SemaphoreType.DMA((2, 2)),
            pltpu.SemaphoreType.DMA((2, 2 * B)),
            pltpu.SemaphoreType.DMA((2, 2 * B)),
        ],
        compiler_params=(pltpu.CompilerParams(collective_id=0)
                         if (do_comm or VARIANT == "barrieronly")
                         else pltpu.CompilerParams()),
    )(x, Wq, K_ext, V_ext, Wo)
